# baseline (speedup 1.0000x reference)
import functools

import jax
import jax.numpy as jnp
from jax import lax
from jax.experimental import pallas as pl
import jax.experimental.pallas.tpu as pltpu
from jax.experimental.pallas import tpu_sc as plsc

E = 64
TOP_K = 2
D_MODEL = 128
BATCH = 64
SEQ_LEN = 4096

S_TC = 2560
S_SC = SEQ_LEN - S_TC

B_BLK = 8
N_BBLKS = BATCH // B_BLK


def _tc_sum_kernel(x_ref, out_ref):
    out_ref[...] = jnp.sum(x_ref[...], axis=1)


NUM_CORES = 2
NUM_SUBCORES = 16
NW = NUM_CORES * NUM_SUBCORES
ROWS_PER_W = BATCH // NW
CH = 384
NCH = S_SC // CH
TOTAL_CH = ROWS_PER_W * NCH
N_VREG = D_MODEL // 16


def _sc_sum_body(x_hbm, out_hbm, buf0, buf1, acc_v, sem0, sem1):
    c = lax.axis_index("c")
    s = lax.axis_index("s")
    w = s * NUM_CORES + c
    row0 = w * ROWS_PER_W
    bufs = (buf0, buf1)
    sems = (sem0, sem1)

    def chunk_src(k):
        r = k // NCH
        cc = k % NCH
        off = (row0 + r) * SEQ_LEN + S_TC + cc * CH
        return x_hbm.at[pl.ds(off, CH)]

    pltpu.make_async_copy(chunk_src(0), bufs[0], sems[0]).start()

    for r in range(ROWS_PER_W):
        accs = tuple(jnp.zeros((16,), jnp.float32) for _ in range(N_VREG))
        for cc in range(NCH):
            k = r * NCH + cc
            if k + 1 < TOTAL_CH:
                kb = (k + 1) % 2
                pltpu.make_async_copy(chunk_src(k + 1), bufs[kb], sems[kb]).start()
            buf = bufs[k % 2]
            pltpu.make_async_copy(chunk_src(k), buf, sems[k % 2]).wait()

            def body(si, a, _buf=buf):
                return tuple(
                    a[j] + _buf[si, j * 16 : (j + 1) * 16] for j in range(N_VREG)
                )

            accs = lax.fori_loop(0, CH, body, accs, unroll=4)
        for j in range(N_VREG):
            acc_v[r, j * 16 : (j + 1) * 16] = accs[j]
    pltpu.sync_copy(acc_v, out_hbm.at[w])


_sc_sum = functools.partial(
    pl.kernel,
    out_type=jax.ShapeDtypeStruct((NW, ROWS_PER_W, D_MODEL), jnp.float32),
    mesh=plsc.VectorSubcoreMesh(
        core_axis_name="c",
        subcore_axis_name="s",
        num_cores=NUM_CORES,
        num_subcores=NUM_SUBCORES,
    ),
    scratch_types=[
        pltpu.VMEM((CH, D_MODEL), jnp.float32),
        pltpu.VMEM((CH, D_MODEL), jnp.float32),
        pltpu.VMEM((ROWS_PER_W, D_MODEL), jnp.float32),
        pltpu.SemaphoreType.DMA,
        pltpu.SemaphoreType.DMA,
    ],
)(_sc_sum_body)


def _combine_kernel(ptc_ref, psc_ref, c_ref, scores_out_ref, idx_out_ref):
    q = (ptc_ref[...] + psc_ref[...]) * (1.0 / SEQ_LEN)
    qn = jnp.sqrt(jnp.sum(q * q, axis=1, keepdims=True))
    q = q / jnp.maximum(qn, 1e-12)

    c = c_ref[...]
    cn = jnp.sqrt(jnp.sum(c * c, axis=1, keepdims=True))
    c = c / jnp.maximum(cn, 1e-12)

    scores = jax.lax.dot_general(
        q, c, (((1,), (1,)), ((), ())), preferred_element_type=jnp.float32
    )

    iota = jax.lax.broadcasted_iota(jnp.int32, (BATCH, E), 1)
    m1 = jnp.max(scores, axis=1, keepdims=True)
    i1 = jnp.min(
        jnp.where(scores == m1, iota, jnp.int32(2**30)), axis=1, keepdims=True
    )
    masked = jnp.where(iota == i1, -jnp.inf, scores)
    m2 = jnp.max(masked, axis=1, keepdims=True)
    i2 = jnp.min(
        jnp.where(masked == m2, iota, jnp.int32(2**30)), axis=1, keepdims=True
    )

    scores_out_ref[:, 0:1] = m1
    scores_out_ref[:, 1:2] = m2
    idx_out_ref[:, 0:1] = i1
    idx_out_ref[:, 1:2] = i2


@jax.jit
def kernel(x, centroids):
    p_sc = _sc_sum(x.reshape(BATCH * SEQ_LEN, D_MODEL))
    p_sc = p_sc.reshape(BATCH, D_MODEL)

    p_tc = pl.pallas_call(
        _tc_sum_kernel,
        grid=(N_BBLKS,),
        in_specs=[pl.BlockSpec((B_BLK, S_TC, D_MODEL), lambda i: (i, 0, 0))],
        out_specs=pl.BlockSpec((B_BLK, D_MODEL), lambda i: (i, 0)),
        out_shape=jax.ShapeDtypeStruct((BATCH, D_MODEL), jnp.float32),
        compiler_params=pltpu.CompilerParams(
            dimension_semantics=("arbitrary",),
        ),
    )(x)

    top_scores, top_idx = pl.pallas_call(
        _combine_kernel,
        out_shape=[
            jax.ShapeDtypeStruct((BATCH, TOP_K), jnp.float32),
            jax.ShapeDtypeStruct((BATCH, TOP_K), jnp.int32),
        ],
    )(p_tc, p_sc, centroids)
    return top_scores, top_idx


# speedup vs baseline: 1.2503x; 1.2503x over previous
import jax
import jax.numpy as jnp
from jax.experimental import pallas as pl
import jax.experimental.pallas.tpu as pltpu

E = 64
TOP_K = 2
D_MODEL = 128
BATCH = 64
SEQ_LEN = 4096

B_BLK = 8
N_BBLKS = BATCH // B_BLK
S_BLK = 1024
N_SBLKS = SEQ_LEN // S_BLK


def _router_kernel(x_ref, c_ref, scores_out_ref, idx_out_ref, acc_ref):
    s = pl.program_id(1)

    @pl.when(s == 0)
    def _init():
        acc_ref[...] = jnp.zeros_like(acc_ref)

    acc_ref[...] += jnp.sum(x_ref[...], axis=1)

    @pl.when(s == N_SBLKS - 1)
    def _finalize():
        q = acc_ref[...] * (1.0 / SEQ_LEN)
        qn = jnp.sqrt(jnp.sum(q * q, axis=1, keepdims=True))
        q = q / jnp.maximum(qn, 1e-12)

        c = c_ref[...]
        cn = jnp.sqrt(jnp.sum(c * c, axis=1, keepdims=True))
        c = c / jnp.maximum(cn, 1e-12)

        scores = jax.lax.dot_general(
            q, c, (((1,), (1,)), ((), ())), preferred_element_type=jnp.float32
        )

        iota = jax.lax.broadcasted_iota(jnp.int32, (B_BLK, E), 1)
        m1 = jnp.max(scores, axis=1, keepdims=True)
        i1 = jnp.min(
            jnp.where(scores == m1, iota, jnp.int32(2**30)), axis=1, keepdims=True
        )
        masked = jnp.where(iota == i1, -jnp.inf, scores)
        m2 = jnp.max(masked, axis=1, keepdims=True)
        i2 = jnp.min(
            jnp.where(masked == m2, iota, jnp.int32(2**30)), axis=1, keepdims=True
        )

        scores_out_ref[:, 0:1] = m1
        scores_out_ref[:, 1:2] = m2
        idx_out_ref[:, 0:1] = i1
        idx_out_ref[:, 1:2] = i2


@jax.jit
def kernel(x, centroids):
    top_scores, top_idx = pl.pallas_call(
        _router_kernel,
        grid=(N_BBLKS, N_SBLKS),
        in_specs=[
            pl.BlockSpec((B_BLK, S_BLK, D_MODEL), lambda i, j: (i, j, 0)),
            pl.BlockSpec((E, D_MODEL), lambda i, j: (0, 0)),
        ],
        out_specs=[
            pl.BlockSpec((B_BLK, TOP_K), lambda i, j: (i, 0)),
            pl.BlockSpec((B_BLK, TOP_K), lambda i, j: (i, 0)),
        ],
        out_shape=[
            jax.ShapeDtypeStruct((BATCH, TOP_K), jnp.float32),
            jax.ShapeDtypeStruct((BATCH, TOP_K), jnp.int32),
        ],
        scratch_shapes=[pltpu.VMEM((B_BLK, D_MODEL), jnp.float32)],
        compiler_params=pltpu.CompilerParams(
            dimension_semantics=("arbitrary", "arbitrary"),
        ),
    )(x, centroids)
    return top_scores, top_idx


# speedup vs baseline: 1.3995x; 1.1193x over previous
import jax
import jax.numpy as jnp
from jax.experimental import pallas as pl
import jax.experimental.pallas.tpu as pltpu

E = 64
TOP_K = 2
D_MODEL = 128
BATCH = 64
SEQ_LEN = 4096

CHUNK_B = 4
N_CH = BATCH // CHUNK_B
RING = 4


def _router_kernel(x_hbm, c_ref, scores_out_ref, idx_out_ref, acc_ref, *rest):
    bufs = rest[:RING]
    sems = rest[RING:]

    def copy(k):
        return pltpu.make_async_copy(
            x_hbm.at[pl.ds(k * CHUNK_B, CHUNK_B)], bufs[k % RING], sems[k % RING]
        )

    for k in range(RING):
        copy(k).start()
    for k in range(N_CH):
        copy(k).wait()
        acc_ref[pl.ds(k * CHUNK_B, CHUNK_B), :] = jnp.sum(bufs[k % RING][...], axis=1)
        if k + RING < N_CH:
            copy(k + RING).start()

    q = acc_ref[...] * (1.0 / SEQ_LEN)
    qn = jnp.sqrt(jnp.sum(q * q, axis=1, keepdims=True))
    q = q / jnp.maximum(qn, 1e-12)

    c = c_ref[...]
    cn = jnp.sqrt(jnp.sum(c * c, axis=1, keepdims=True))
    c = c / jnp.maximum(cn, 1e-12)

    scores = jax.lax.dot_general(
        q, c, (((1,), (1,)), ((), ())), preferred_element_type=jnp.float32
    )

    iota = jax.lax.broadcasted_iota(jnp.int32, (BATCH, E), 1)
    m1 = jnp.max(scores, axis=1, keepdims=True)
    i1 = jnp.min(
        jnp.where(scores == m1, iota, jnp.int32(2**30)), axis=1, keepdims=True
    )
    masked = jnp.where(iota == i1, -jnp.inf, scores)
    m2 = jnp.max(masked, axis=1, keepdims=True)
    i2 = jnp.min(
        jnp.where(masked == m2, iota, jnp.int32(2**30)), axis=1, keepdims=True
    )

    scores_out_ref[:, 0:1] = m1
    scores_out_ref[:, 1:2] = m2
    idx_out_ref[:, 0:1] = i1
    idx_out_ref[:, 1:2] = i2


@jax.jit
def kernel(x, centroids):
    top_scores, top_idx = pl.pallas_call(
        _router_kernel,
        in_specs=[
            pl.BlockSpec(memory_space=pl.ANY),
            pl.BlockSpec(memory_space=pltpu.MemorySpace.VMEM),
        ],
        out_specs=[
            pl.BlockSpec(memory_space=pltpu.MemorySpace.VMEM),
            pl.BlockSpec(memory_space=pltpu.MemorySpace.VMEM),
        ],
        out_shape=[
            jax.ShapeDtypeStruct((BATCH, TOP_K), jnp.float32),
            jax.ShapeDtypeStruct((BATCH, TOP_K), jnp.int32),
        ],
        scratch_shapes=(
            [pltpu.VMEM((BATCH, D_MODEL), jnp.float32)]
            + [pltpu.VMEM((CHUNK_B, SEQ_LEN, D_MODEL), jnp.float32) for _ in range(RING)]
            + [pltpu.SemaphoreType.DMA for _ in range(RING)]
        ),
    )(x, centroids)
    return top_scores, top_idx
